# baseline (speedup 1.0000x reference)
import functools

import jax
import jax.numpy as jnp
from jax import lax
from jax.experimental import pallas as pl
from jax.experimental.pallas import tpu as pltpu
from jax.experimental.pallas import tpu_sc as plsc

H = 128
G = 64
NC = 2
NS = 16
NW = NC * NS
CH = 128


def _deg_body(n_pad, cpw, dst_hbm, ones_hbm, zeros_hbm, out_hbm,
              dst_v, ones_v, acc):
    c = lax.axis_index("c")
    s = lax.axis_index("s")
    wid = c * NS + s
    rpt = n_pad // NS
    pltpu.sync_copy(zeros_hbm.at[pl.ds(s * rpt, rpt)],
                    acc.at[pl.ds(s * rpt, rpt)])
    pltpu.sync_copy(dst_hbm.at[pl.ds(wid * cpw, cpw)], dst_v)
    pltpu.sync_copy(ones_hbm, ones_v)
    plsc.subcore_barrier()

    def body(j, carry):
        pltpu.sync_copy(ones_v, acc.at[dst_v.at[j]], add=True)
        return carry

    lax.fori_loop(0, cpw, body, 0)
    plsc.subcore_barrier()
    pltpu.sync_copy(acc.at[pl.ds(s * rpt, rpt)],
                    out_hbm.at[c, pl.ds(s * rpt, rpt)])


WB = 8


def _aggr_body(n_pad, cpw, src_hbm, dst_hbm, g_hbm, zeros_hbm,
               out_hbm, swin, dwin, rows, psem, acc):
    c = lax.axis_index("c")
    s = lax.axis_index("s")
    wid = c * NS + s
    rpt = n_pad // NS
    pltpu.sync_copy(zeros_hbm.at[pl.ds(s * rpt, rpt)],
                    acc.at[pl.ds(s * rpt, rpt)])
    plsc.subcore_barrier()

    nwin = cpw // WB

    def win(w, carry):
        base = wid * cpw + w * WB
        pltpu.sync_copy(src_hbm.at[pl.ds(base, WB)], swin)
        pltpu.sync_copy(dst_hbm.at[pl.ds(base, WB)], dwin)
        for k in range(WB):
            pltpu.async_copy(g_hbm.at[swin.at[k]], rows, psem).wait()
            pltpu.sync_copy(rows, acc.at[dwin.at[k]], add=True)
        return carry

    lax.fori_loop(0, nwin, win, 0)
    plsc.subcore_barrier()
    pltpu.sync_copy(acc.at[pl.ds(s * rpt, rpt)],
                    out_hbm.at[c, pl.ds(s * rpt, rpt)])


def _dense1_body(deg2_ref, emb_ref, w1a_ref, b1a_ref, w1b_ref, b1b_ref,
                 g_ref):
    d = deg2_ref[0, :, 0:1] + deg2_ref[1, :, 0:1]
    u = jnp.dot(emb_ref[...], w1a_ref[...],
                preferred_element_type=jnp.float32)
    t = jnp.maximum((1.0 + d) * u + b1a_ref[...], 0.0)
    h1 = jnp.dot(t, w1b_ref[...],
                 preferred_element_type=jnp.float32) + b1b_ref[...]
    g_ref[...] = jnp.maximum(h1, 0.0)


def _dense2_body(g_ref, p_ref, batch_ref, w2a_ref, b2a_ref, w2b_ref,
                 b2b_ref, wlin_ref, blin_ref, out_ref):
    z = g_ref[...] + p_ref[0] + p_ref[1]
    t = jnp.maximum(
        jnp.dot(z, w2a_ref[...], preferred_element_type=jnp.float32)
        + b2a_ref[...], 0.0)
    h2 = jnp.dot(t, w2b_ref[...],
                 preferred_element_type=jnp.float32) + b2b_ref[...]
    gid = lax.broadcasted_iota(jnp.int32, (G, batch_ref.shape[1]), 0)
    m = (gid == batch_ref[...]).astype(jnp.float32)
    sums = jnp.dot(m, h2, preferred_element_type=jnp.float32)
    counts = jnp.sum(m, axis=1, keepdims=True)
    pooled = sums / jnp.maximum(counts, 1.0)
    out_ref[...] = jnp.dot(pooled, wlin_ref[...],
                           preferred_element_type=jnp.float32) + blin_ref[...]


def kernel(x, edge_index, edge_attr, batch, emb, W1a, b1a, W1b, b1b,
           W2a, b2a, W2b, b2b, Wlin, blin):
    n = x.shape[0]
    e = edge_index.shape[1]
    n_pad = ((n + NS * 8 - 1) // (NS * 8)) * (NS * 8)
    cpw = (e + NW * CH - 1) // (NW * CH)
    cpw = ((cpw + 7) // 8) * 8
    e_pad = NW * CH * cpw
    pad_idx = n

    src_p = jnp.concatenate(
        [edge_index[0], jnp.full((e_pad - e,), pad_idx, jnp.int32)]
    ).reshape(NW * cpw, CH)
    dst_p = jnp.concatenate(
        [edge_index[1], jnp.full((e_pad - e,), pad_idx, jnp.int32)]
    ).reshape(NW * cpw, CH)

    ones_h = jnp.ones((CH, H), jnp.float32)
    zeros_h = jnp.zeros((n_pad, H), jnp.float32)

    mesh = plsc.VectorSubcoreMesh(
        core_axis_name="c", subcore_axis_name="s",
        num_cores=NC, num_subcores=NS)

    deg_call = pl.kernel(
        functools.partial(_deg_body, n_pad, cpw),
        out_type=jax.ShapeDtypeStruct((NC, n_pad, H), jnp.float32),
        mesh=mesh,
        scratch_types=[
            pltpu.VMEM((cpw, CH), jnp.int32),
            pltpu.VMEM((CH, H), jnp.float32),
            pltpu.VMEM_SHARED((n_pad, H), jnp.float32),
        ],
    )
    deg2 = deg_call(dst_p, ones_h, zeros_h)

    g = pl.pallas_call(
        _dense1_body,
        out_shape=jax.ShapeDtypeStruct((n_pad, H), jnp.float32),
    )(deg2, emb, W1a, b1a[None], W1b, b1b[None])

    aggr_call = pl.kernel(
        functools.partial(_aggr_body, n_pad, cpw),
        out_type=jax.ShapeDtypeStruct((NC, n_pad, H), jnp.float32),
        mesh=mesh,
        scratch_types=[
            pltpu.VMEM((WB, CH), jnp.int32),
            pltpu.VMEM((WB, CH), jnp.int32),
            pltpu.VMEM((CH, H), jnp.float32),
            pltpu.SemaphoreType.DMA,
            pltpu.VMEM_SHARED((n_pad, H), jnp.float32),
        ],
    )
    parts = aggr_call(src_p, dst_p, g, zeros_h)

    batch_p = jnp.concatenate(
        [batch, jnp.full((n_pad - n,), -1, jnp.int32)])[None]

    out = pl.pallas_call(
        _dense2_body,
        out_shape=jax.ShapeDtypeStruct((G, Wlin.shape[1]), jnp.float32),
    )(g, parts, batch_p, W2a, b2a[None], W2b, b2b[None], Wlin, blin[None])
    return out


# speedup vs baseline: 1.0886x; 1.0886x over previous
import functools

import jax
import jax.numpy as jnp
from jax import lax
from jax.experimental import pallas as pl
from jax.experimental.pallas import tpu as pltpu
from jax.experimental.pallas import tpu_sc as plsc

H = 128
G = 64
NC = 2
NS = 16
NW = NC * NS
CH = 128


def _deg_body(n_pad, cpw, dst_hbm, ones_hbm, zeros_hbm, out_hbm,
              dst_v, ones_v, acc):
    c = lax.axis_index("c")
    s = lax.axis_index("s")
    wid = c * NS + s
    rpt = n_pad // NS
    pltpu.sync_copy(zeros_hbm.at[pl.ds(s * rpt, rpt)],
                    acc.at[pl.ds(s * rpt, rpt)])
    pltpu.sync_copy(dst_hbm.at[pl.ds(wid * cpw, cpw)], dst_v)
    pltpu.sync_copy(ones_hbm, ones_v)
    plsc.subcore_barrier()

    def body(j, carry):
        pltpu.sync_copy(ones_v, acc.at[dst_v.at[j]], add=True)
        return carry

    lax.fori_loop(0, cpw, body, 0)
    plsc.subcore_barrier()
    pltpu.sync_copy(acc.at[pl.ds(s * rpt, rpt)],
                    out_hbm.at[c, pl.ds(s * rpt, rpt)])


WB = 8


def _aggr_body(n_pad, cpw, src_hbm, dst_hbm, g_hbm, zeros_hbm,
               out_hbm, swin, dwin, rows0, rows1, psem0, psem1, acc):
    c = lax.axis_index("c")
    s = lax.axis_index("s")
    wid = c * NS + s
    rpt = n_pad // NS
    pltpu.sync_copy(zeros_hbm.at[pl.ds(s * rpt, rpt)],
                    acc.at[pl.ds(s * rpt, rpt)])
    plsc.subcore_barrier()

    nwin = cpw // WB

    bufs = (rows0, rows1)
    sems = (psem0, psem1)

    def win(w, carry):
        base = wid * cpw + w * WB
        pltpu.sync_copy(src_hbm.at[pl.ds(base, WB)], swin)
        pltpu.sync_copy(dst_hbm.at[pl.ds(base, WB)], dwin)
        cp = pltpu.async_copy(g_hbm.at[swin.at[0]], bufs[0], sems[0])
        for k in range(WB):
            nxt = None
            if k + 1 < WB:
                nxt = pltpu.async_copy(g_hbm.at[swin.at[k + 1]],
                                       bufs[(k + 1) % 2], sems[(k + 1) % 2])
            cp.wait()
            pltpu.sync_copy(bufs[k % 2], acc.at[dwin.at[k]], add=True)
            cp = nxt
        return carry

    lax.fori_loop(0, nwin, win, 0)
    plsc.subcore_barrier()
    pltpu.sync_copy(acc.at[pl.ds(s * rpt, rpt)],
                    out_hbm.at[c, pl.ds(s * rpt, rpt)])


def _dense1_body(deg2_ref, emb_ref, w1a_ref, b1a_ref, w1b_ref, b1b_ref,
                 g_ref):
    d = deg2_ref[0, :, 0:1] + deg2_ref[1, :, 0:1]
    u = jnp.dot(emb_ref[...], w1a_ref[...],
                preferred_element_type=jnp.float32)
    t = jnp.maximum((1.0 + d) * u + b1a_ref[...], 0.0)
    h1 = jnp.dot(t, w1b_ref[...],
                 preferred_element_type=jnp.float32) + b1b_ref[...]
    g_ref[...] = jnp.maximum(h1, 0.0)


def _dense2_body(g_ref, p_ref, batch_ref, w2a_ref, b2a_ref, w2b_ref,
                 b2b_ref, wlin_ref, blin_ref, out_ref):
    z = g_ref[...] + p_ref[0] + p_ref[1]
    t = jnp.maximum(
        jnp.dot(z, w2a_ref[...], preferred_element_type=jnp.float32)
        + b2a_ref[...], 0.0)
    h2 = jnp.dot(t, w2b_ref[...],
                 preferred_element_type=jnp.float32) + b2b_ref[...]
    gid = lax.broadcasted_iota(jnp.int32, (G, batch_ref.shape[1]), 0)
    m = (gid == batch_ref[...]).astype(jnp.float32)
    sums = jnp.dot(m, h2, preferred_element_type=jnp.float32)
    counts = jnp.sum(m, axis=1, keepdims=True)
    pooled = sums / jnp.maximum(counts, 1.0)
    out_ref[...] = jnp.dot(pooled, wlin_ref[...],
                           preferred_element_type=jnp.float32) + blin_ref[...]


def kernel(x, edge_index, edge_attr, batch, emb, W1a, b1a, W1b, b1b,
           W2a, b2a, W2b, b2b, Wlin, blin):
    n = x.shape[0]
    e = edge_index.shape[1]
    n_pad = ((n + NS * 8 - 1) // (NS * 8)) * (NS * 8)
    cpw = (e + NW * CH - 1) // (NW * CH)
    cpw = ((cpw + 7) // 8) * 8
    e_pad = NW * CH * cpw
    pad_idx = n

    src_p = jnp.concatenate(
        [edge_index[0], jnp.full((e_pad - e,), pad_idx, jnp.int32)]
    ).reshape(NW * cpw, CH)
    dst_p = jnp.concatenate(
        [edge_index[1], jnp.full((e_pad - e,), pad_idx, jnp.int32)]
    ).reshape(NW * cpw, CH)

    ones_h = jnp.ones((CH, H), jnp.float32)
    zeros_h = jnp.zeros((n_pad, H), jnp.float32)

    mesh = plsc.VectorSubcoreMesh(
        core_axis_name="c", subcore_axis_name="s",
        num_cores=NC, num_subcores=NS)

    deg_call = pl.kernel(
        functools.partial(_deg_body, n_pad, cpw),
        out_type=jax.ShapeDtypeStruct((NC, n_pad, H), jnp.float32),
        mesh=mesh,
        scratch_types=[
            pltpu.VMEM((cpw, CH), jnp.int32),
            pltpu.VMEM((CH, H), jnp.float32),
            pltpu.VMEM_SHARED((n_pad, H), jnp.float32),
        ],
    )
    deg2 = deg_call(dst_p, ones_h, zeros_h)

    g = pl.pallas_call(
        _dense1_body,
        out_shape=jax.ShapeDtypeStruct((n_pad, H), jnp.float32),
    )(deg2, emb, W1a, b1a[None], W1b, b1b[None])

    aggr_call = pl.kernel(
        functools.partial(_aggr_body, n_pad, cpw),
        out_type=jax.ShapeDtypeStruct((NC, n_pad, H), jnp.float32),
        mesh=mesh,
        scratch_types=[
            pltpu.VMEM((WB, CH), jnp.int32),
            pltpu.VMEM((WB, CH), jnp.int32),
            pltpu.VMEM((CH, H), jnp.float32),
            pltpu.VMEM((CH, H), jnp.float32),
            pltpu.SemaphoreType.DMA,
            pltpu.SemaphoreType.DMA,
            pltpu.VMEM_SHARED((n_pad, H), jnp.float32),
        ],
    )
    parts = aggr_call(src_p, dst_p, g, zeros_h)

    batch_p = jnp.concatenate(
        [batch, jnp.full((n_pad - n,), -1, jnp.int32)])[None]

    out = pl.pallas_call(
        _dense2_body,
        out_shape=jax.ShapeDtypeStruct((G, Wlin.shape[1]), jnp.float32),
    )(g, parts, batch_p, W2a, b2a[None], W2b, b2b[None], Wlin, blin[None])
    return out


# speedup vs baseline: 1.1135x; 1.0228x over previous
import functools

import jax
import jax.numpy as jnp
from jax import lax
from jax.experimental import pallas as pl
from jax.experimental.pallas import tpu as pltpu
from jax.experimental.pallas import tpu_sc as plsc

H = 128
G = 64
NC = 2
NS = 16
NW = NC * NS
CH = 128


def _deg_body(n_pad, cpw, dst_hbm, ones_hbm, zeros_hbm, out_hbm,
              dst_v, ones_v, acc):
    c = lax.axis_index("c")
    s = lax.axis_index("s")
    wid = c * NS + s
    rpt = n_pad // NS
    pltpu.sync_copy(zeros_hbm.at[pl.ds(s * rpt, rpt)],
                    acc.at[pl.ds(s * rpt, rpt)])
    pltpu.sync_copy(dst_hbm.at[pl.ds(wid * cpw, cpw)], dst_v)
    pltpu.sync_copy(ones_hbm, ones_v)
    plsc.subcore_barrier()

    def body(j, carry):
        pltpu.sync_copy(ones_v, acc.at[dst_v.at[j]], add=True)
        return carry

    lax.fori_loop(0, cpw, body, 0)
    plsc.subcore_barrier()
    pltpu.sync_copy(acc.at[pl.ds(s * rpt, rpt)],
                    out_hbm.at[c, pl.ds(s * rpt, rpt)])


WB = 40


def _aggr_body(n_pad, cpw, src_hbm, dst_hbm, g_hbm, zeros_hbm,
               out_hbm, swin, dwin, rows0, rows1, psem0, psem1, acc):
    c = lax.axis_index("c")
    s = lax.axis_index("s")
    wid = c * NS + s
    rpt = n_pad // NS
    pltpu.sync_copy(zeros_hbm.at[pl.ds(s * rpt, rpt)],
                    acc.at[pl.ds(s * rpt, rpt)])
    plsc.subcore_barrier()

    nwin = cpw // WB

    bufs = (rows0, rows1)
    sems = (psem0, psem1)

    def win(w, carry):
        base = wid * cpw + w * WB
        pltpu.sync_copy(src_hbm.at[pl.ds(base, WB)], swin)
        pltpu.sync_copy(dst_hbm.at[pl.ds(base, WB)], dwin)
        cp = pltpu.async_copy(g_hbm.at[swin.at[0]], bufs[0], sems[0])
        for k in range(WB):
            nxt = None
            if k + 1 < WB:
                nxt = pltpu.async_copy(g_hbm.at[swin.at[k + 1]],
                                       bufs[(k + 1) % 2], sems[(k + 1) % 2])
            cp.wait()
            pltpu.sync_copy(bufs[k % 2], acc.at[dwin.at[k]], add=True)
            cp = nxt
        return carry

    lax.fori_loop(0, nwin, win, 0)
    plsc.subcore_barrier()
    pltpu.sync_copy(acc.at[pl.ds(s * rpt, rpt)],
                    out_hbm.at[c, pl.ds(s * rpt, rpt)])


def _dense1_body(deg2_ref, emb_ref, w1a_ref, b1a_ref, w1b_ref, b1b_ref,
                 g_ref):
    d = deg2_ref[0, :, 0:1] + deg2_ref[1, :, 0:1]
    u = jnp.dot(emb_ref[...], w1a_ref[...],
                preferred_element_type=jnp.float32)
    t = jnp.maximum((1.0 + d) * u + b1a_ref[...], 0.0)
    h1 = jnp.dot(t, w1b_ref[...],
                 preferred_element_type=jnp.float32) + b1b_ref[...]
    g_ref[...] = jnp.maximum(h1, 0.0)


def _dense2_body(g_ref, p_ref, batch_ref, w2a_ref, b2a_ref, w2b_ref,
                 b2b_ref, wlin_ref, blin_ref, out_ref):
    z = g_ref[...] + p_ref[0] + p_ref[1]
    t = jnp.maximum(
        jnp.dot(z, w2a_ref[...], preferred_element_type=jnp.float32)
        + b2a_ref[...], 0.0)
    h2 = jnp.dot(t, w2b_ref[...],
                 preferred_element_type=jnp.float32) + b2b_ref[...]
    gid = lax.broadcasted_iota(jnp.int32, (G, batch_ref.shape[1]), 0)
    m = (gid == batch_ref[...]).astype(jnp.float32)
    sums = jnp.dot(m, h2, preferred_element_type=jnp.float32)
    counts = jnp.sum(m, axis=1, keepdims=True)
    pooled = sums / jnp.maximum(counts, 1.0)
    out_ref[...] = jnp.dot(pooled, wlin_ref[...],
                           preferred_element_type=jnp.float32) + blin_ref[...]


def kernel(x, edge_index, edge_attr, batch, emb, W1a, b1a, W1b, b1b,
           W2a, b2a, W2b, b2b, Wlin, blin):
    n = x.shape[0]
    e = edge_index.shape[1]
    n_pad = ((n + NS * 8 - 1) // (NS * 8)) * (NS * 8)
    cpw = (e + NW * CH - 1) // (NW * CH)
    cpw = ((cpw + WB - 1) // WB) * WB
    e_pad = NW * CH * cpw
    pad_idx = n

    src_p = jnp.concatenate(
        [edge_index[0], jnp.full((e_pad - e,), pad_idx, jnp.int32)]
    ).reshape(NW * cpw, CH)
    dst_p = jnp.concatenate(
        [edge_index[1], jnp.full((e_pad - e,), pad_idx, jnp.int32)]
    ).reshape(NW * cpw, CH)

    ones_h = jnp.ones((CH, H), jnp.float32)
    zeros_h = jnp.zeros((n_pad, H), jnp.float32)

    mesh = plsc.VectorSubcoreMesh(
        core_axis_name="c", subcore_axis_name="s",
        num_cores=NC, num_subcores=NS)

    deg_call = pl.kernel(
        functools.partial(_deg_body, n_pad, cpw),
        out_type=jax.ShapeDtypeStruct((NC, n_pad, H), jnp.float32),
        mesh=mesh,
        scratch_types=[
            pltpu.VMEM((cpw, CH), jnp.int32),
            pltpu.VMEM((CH, H), jnp.float32),
            pltpu.VMEM_SHARED((n_pad, H), jnp.float32),
        ],
    )
    deg2 = deg_call(dst_p, ones_h, zeros_h)

    g = pl.pallas_call(
        _dense1_body,
        out_shape=jax.ShapeDtypeStruct((n_pad, H), jnp.float32),
    )(deg2, emb, W1a, b1a[None], W1b, b1b[None])

    aggr_call = pl.kernel(
        functools.partial(_aggr_body, n_pad, cpw),
        out_type=jax.ShapeDtypeStruct((NC, n_pad, H), jnp.float32),
        mesh=mesh,
        scratch_types=[
            pltpu.VMEM((WB, CH), jnp.int32),
            pltpu.VMEM((WB, CH), jnp.int32),
            pltpu.VMEM((CH, H), jnp.float32),
            pltpu.VMEM((CH, H), jnp.float32),
            pltpu.SemaphoreType.DMA,
            pltpu.SemaphoreType.DMA,
            pltpu.VMEM_SHARED((n_pad, H), jnp.float32),
        ],
    )
    parts = aggr_call(src_p, dst_p, g, zeros_h)

    batch_p = jnp.concatenate(
        [batch, jnp.full((n_pad - n,), -1, jnp.int32)])[None]

    out = pl.pallas_call(
        _dense2_body,
        out_shape=jax.ShapeDtypeStruct((G, Wlin.shape[1]), jnp.float32),
    )(g, parts, batch_p, W2a, b2a[None], W2b, b2b[None], Wlin, blin[None])
    return out
